# baseline (speedup 1.0000x reference)
import functools

import jax
import jax.numpy as jnp
from jax.experimental import pallas as pl
from jax.experimental.pallas import tpu as pltpu


def _bitonic_sort_kernel(x_ref, o_ref, *, n: int):
    x = x_ref[0]
    L = x.shape[1]
    log_n = n.bit_length() - 1
    idx = jax.lax.broadcasted_iota(jnp.int32, (n, L), 0)
    zero = jnp.zeros((n, L), dtype=jnp.int32)
    for k in range(1, log_n + 1):
        for j in range(k - 1, -1, -1):
            d = 1 << j
            if d >= 8:
                G = n // (2 * d)
                v = x.reshape(G, 2, d, L)
                a = v[:, 0]
                b = v[:, 1]
                mn = jnp.minimum(a, b)
                mx = jnp.maximum(a, b)
                if k == log_n:
                    na, nb = mn, mx
                else:
                    g = jax.lax.broadcasted_iota(jnp.int32, (G, 1, 1), 0)
                    asc = (g & (1 << (k - j - 1))) == jnp.zeros(
                        (G, 1, 1), dtype=jnp.int32)
                    na = jnp.where(asc, mn, mx)
                    nb = jnp.where(asc, mx, mn)
                x = jnp.concatenate(
                    [na[:, None], nb[:, None]], axis=1).reshape(n, L)
            else:
                upper = (idx & d) != zero
                asc = (idx & (1 << k)) == zero
                keep_min = upper != asc
                fwd = pltpu.roll(x, n - d, 0)
                bwd = pltpu.roll(x, d, 0)
                partner = jnp.where(upper, bwd, fwd)
                x = jnp.where(keep_min, jnp.minimum(x, partner),
                              jnp.maximum(x, partner))
    o_ref[0] = x


@jax.jit
def kernel(x):
    b, n, f = x.shape
    lane_tile = 128
    grid = (b, f // lane_tile)
    return pl.pallas_call(
        functools.partial(_bitonic_sort_kernel, n=n),
        grid=grid,
        in_specs=[pl.BlockSpec((1, n, lane_tile), lambda i, j: (i, 0, j))],
        out_specs=pl.BlockSpec((1, n, lane_tile), lambda i, j: (i, 0, j)),
        out_shape=jax.ShapeDtypeStruct(x.shape, x.dtype),
        compiler_params=pltpu.CompilerParams(
            dimension_semantics=("parallel", "parallel"),
        ),
    )(x)


# speedup vs baseline: 1.3041x; 1.3041x over previous
import functools

import jax
import jax.numpy as jnp
from jax.experimental import pallas as pl
from jax.experimental.pallas import tpu as pltpu

_C = 64
_LOGC = 6


def _ce_chunk(v, k, j, asc_scalar):
    C, L = v.shape
    d = 1 << j
    if d >= 8:
        G = C // (2 * d)
        w = v.reshape(G, 2, d, L)
        a = w[:, 0]
        b = w[:, 1]
        mn = jnp.minimum(a, b)
        mx = jnp.maximum(a, b)
        if asc_scalar is None:
            g = jax.lax.broadcasted_iota(jnp.int32, (G, 1, 1), 0)
            asc = ((g >> (k - j - 1)) & 1) == jnp.zeros((G, 1, 1), jnp.int32)
        else:
            asc = asc_scalar
        na = jnp.where(asc, mn, mx)
        nb = jnp.where(asc, mx, mn)
        return jnp.concatenate([na[:, None], nb[:, None]], axis=1).reshape(
            C, L)
    il = jax.lax.broadcasted_iota(jnp.int32, (C, L), 0)
    zero = jnp.zeros((C, L), jnp.int32)
    upper = (il & d) != zero
    if asc_scalar is None:
        asc = (il & (1 << k)) == zero
        keep_min = upper != asc
    else:
        keep_min = upper != asc_scalar
    fwd = pltpu.roll(v, C - d, 0)
    bwd = pltpu.roll(v, d, 0)
    partner = jnp.where(upper, bwd, fwd)
    return jnp.where(keep_min, jnp.minimum(v, partner),
                     jnp.maximum(v, partner))


def _bitonic_sort_kernel(x_ref, o_ref, *, n: int):
    log_n = n.bit_length() - 1
    nc = n // _C

    def phase_a(c, carry):
        v = x_ref[0, pl.ds(c * _C, _C), :]
        asc6 = (c & 1) == 0
        for k in range(1, _LOGC + 1):
            for j in range(k - 1, -1, -1):
                v = _ce_chunk(v, k, j, asc6 if k == _LOGC else None)
        o_ref[0, pl.ds(c * _C, _C), :] = v
        return carry

    jax.lax.fori_loop(0, nc, phase_a, 0)

    for k in range(_LOGC + 1, log_n + 1):
        for j in range(k - 1, _LOGC - 1, -1):
            d = 1 << j
            logm = j - _LOGC

            def big(t, carry, k=k, j=j, d=d, logm=logm):
                g = t >> logm
                p = t - (g << logm)
                lo = (g << (j + 1)) + p * _C
                hi = lo + d
                a = o_ref[0, pl.ds(lo, _C), :]
                b = o_ref[0, pl.ds(hi, _C), :]
                mn = jnp.minimum(a, b)
                mx = jnp.maximum(a, b)
                if k == log_n:
                    o_ref[0, pl.ds(lo, _C), :] = mn
                    o_ref[0, pl.ds(hi, _C), :] = mx
                else:
                    asc = ((g >> (k - j - 1)) & 1) == 0
                    o_ref[0, pl.ds(lo, _C), :] = jnp.where(asc, mn, mx)
                    o_ref[0, pl.ds(hi, _C), :] = jnp.where(asc, mx, mn)
                return carry

            jax.lax.fori_loop(0, n // (2 * _C), big, 0)

        def tail(c, carry, k=k):
            v = o_ref[0, pl.ds(c * _C, _C), :]
            asc = ((c >> (k - _LOGC)) & 1) == 0
            for j in range(_LOGC - 1, -1, -1):
                v = _ce_chunk(v, k, j, asc)
            o_ref[0, pl.ds(c * _C, _C), :] = v
            return carry

        jax.lax.fori_loop(0, nc, tail, 0)


@jax.jit
def kernel(x):
    b, n, f = x.shape
    lane_tile = 128
    grid = (b, f // lane_tile)
    return pl.pallas_call(
        functools.partial(_bitonic_sort_kernel, n=n),
        grid=grid,
        in_specs=[pl.BlockSpec((1, n, lane_tile), lambda i, j: (i, 0, j))],
        out_specs=pl.BlockSpec((1, n, lane_tile), lambda i, j: (i, 0, j)),
        out_shape=jax.ShapeDtypeStruct(x.shape, x.dtype),
        compiler_params=pltpu.CompilerParams(
            dimension_semantics=("parallel", "parallel"),
        ),
    )(x)


# speedup vs baseline: 1.4990x; 1.1494x over previous
import functools

import jax
import jax.numpy as jnp
from jax.experimental import pallas as pl
from jax.experimental.pallas import tpu as pltpu

_C = 64
_LOGC = 6


def _ce_masked(v, k, j):
    C, L = v.shape
    d = 1 << j
    if d >= 8:
        G = C // (2 * d)
        w = v.reshape(G, 2, d, L)
        a = w[:, 0]
        b = w[:, 1]
        mn = jnp.minimum(a, b)
        mx = jnp.maximum(a, b)
        g = jax.lax.broadcasted_iota(jnp.int32, (G, 1, 1), 0)
        asc = ((g >> (k - j - 1)) & 1) == jnp.zeros((G, 1, 1), jnp.int32)
        na = jnp.where(asc, mn, mx)
        nb = jnp.where(asc, mx, mn)
        return jnp.concatenate([na[:, None], nb[:, None]], axis=1).reshape(
            C, L)
    il = jax.lax.broadcasted_iota(jnp.int32, (C, L), 0)
    zero = jnp.zeros((C, L), jnp.int32)
    upper = (il & d) != zero
    asc = (il & (1 << k)) == zero
    keep_min = upper != asc
    fwd = pltpu.roll(v, C - d, 0)
    bwd = pltpu.roll(v, d, 0)
    partner = jnp.where(upper, bwd, fwd)
    return jnp.where(keep_min, jnp.minimum(v, partner),
                     jnp.maximum(v, partner))


def _ce_static(v, j, ascending):
    C, L = v.shape
    d = 1 << j
    if d >= 8:
        G = C // (2 * d)
        w = v.reshape(G, 2, d, L)
        a = w[:, 0]
        b = w[:, 1]
        mn = jnp.minimum(a, b)
        mx = jnp.maximum(a, b)
        na, nb = (mn, mx) if ascending else (mx, mn)
        return jnp.concatenate([na[:, None], nb[:, None]], axis=1).reshape(
            C, L)
    il = jax.lax.broadcasted_iota(jnp.int32, (C, L), 0)
    upper = (il & d) != jnp.zeros((C, L), jnp.int32)
    fwd = pltpu.roll(v, C - d, 0)
    bwd = pltpu.roll(v, d, 0)
    if ascending:
        return jnp.where(upper, jnp.maximum(v, bwd), jnp.minimum(v, fwd))
    return jnp.where(upper, jnp.minimum(v, bwd), jnp.maximum(v, fwd))


def _merge_tail(v, ascending):
    for j in range(_LOGC - 1, -1, -1):
        v = _ce_static(v, j, ascending)
    return v


def _bitonic_sort_kernel(x_ref, o_ref, *, n: int):
    log_n = n.bit_length() - 1
    nc = n // _C

    def phase_a(c, carry):
        v = x_ref[0, pl.ds(c * _C, _C), :]
        for k in range(1, _LOGC):
            for j in range(k - 1, -1, -1):
                v = _ce_masked(v, k, j)
        asc6 = (c & 1) == 0

        @pl.when(asc6)
        def _():
            o_ref[0, pl.ds(c * _C, _C), :] = _merge_tail(v, True)

        @pl.when(jnp.logical_not(asc6))
        def _():
            o_ref[0, pl.ds(c * _C, _C), :] = _merge_tail(v, False)

        return carry

    jax.lax.fori_loop(0, nc, phase_a, 0)

    cq = _C // 2

    for k in range(_LOGC + 1, log_n + 1):
        js = list(range(k - 1, _LOGC - 1, -1))
        fused = [(js[i], js[i + 1]) for i in range(0, len(js) - 1, 2)]
        single = js[-1] if len(js) % 2 else None

        for j, j2 in fused:
            d = 1 << j
            d2 = 1 << j2
            logm = j2 - 5

            def big2(t, carry, k=k, j=j, d=d, d2=d2, logm=logm):
                g = t >> logm
                p = t - (g << logm)
                base = (g << (j + 1)) + p * cq
                a0 = o_ref[0, pl.ds(base, cq), :]
                a1 = o_ref[0, pl.ds(base + d2, cq), :]
                a2 = o_ref[0, pl.ds(base + d, cq), :]
                a3 = o_ref[0, pl.ds(base + d + d2, cq), :]
                mn02 = jnp.minimum(a0, a2)
                mx02 = jnp.maximum(a0, a2)
                mn13 = jnp.minimum(a1, a3)
                mx13 = jnp.maximum(a1, a3)
                o0 = jnp.minimum(mn02, mn13)
                o1 = jnp.maximum(mn02, mn13)
                o2 = jnp.minimum(mx02, mx13)
                o3 = jnp.maximum(mx02, mx13)
                if k == log_n:
                    s0, s1, s2, s3 = 0, d2, d, d + d2
                else:
                    asc = ((g >> (k - j - 1)) & 1) == 0
                    s0 = jnp.where(asc, 0, d + d2)
                    s1 = jnp.where(asc, d2, d)
                    s2 = jnp.where(asc, d, d2)
                    s3 = jnp.where(asc, d + d2, 0)
                o_ref[0, pl.ds(base + s0, cq), :] = o0
                o_ref[0, pl.ds(base + s1, cq), :] = o1
                o_ref[0, pl.ds(base + s2, cq), :] = o2
                o_ref[0, pl.ds(base + s3, cq), :] = o3
                return carry

            jax.lax.fori_loop(0, n // (4 * cq), big2, 0)

        if single is not None:
            j = single
            d = 1 << j
            logm = j - _LOGC

            def big1(t, carry, k=k, j=j, d=d, logm=logm):
                g = t >> logm
                p = t - (g << logm)
                lo = (g << (j + 1)) + p * _C
                a = o_ref[0, pl.ds(lo, _C), :]
                b = o_ref[0, pl.ds(lo + d, _C), :]
                mn = jnp.minimum(a, b)
                mx = jnp.maximum(a, b)
                if k == log_n:
                    smn, smx = 0, d
                else:
                    asc = ((g >> (k - j - 1)) & 1) == 0
                    smn = jnp.where(asc, 0, d)
                    smx = jnp.where(asc, d, 0)
                o_ref[0, pl.ds(lo + smn, _C), :] = mn
                o_ref[0, pl.ds(lo + smx, _C), :] = mx
                return carry

            jax.lax.fori_loop(0, n // (2 * _C), big1, 0)

        def tail(c, carry, k=k):
            v = o_ref[0, pl.ds(c * _C, _C), :]
            if k == log_n:
                o_ref[0, pl.ds(c * _C, _C), :] = _merge_tail(v, True)
            else:
                asc = ((c >> (k - _LOGC)) & 1) == 0

                @pl.when(asc)
                def _():
                    o_ref[0, pl.ds(c * _C, _C), :] = _merge_tail(v, True)

                @pl.when(jnp.logical_not(asc))
                def _():
                    o_ref[0, pl.ds(c * _C, _C), :] = _merge_tail(v, False)

            return carry

        jax.lax.fori_loop(0, nc, tail, 0)


@jax.jit
def kernel(x):
    b, n, f = x.shape
    lane_tile = 128
    grid = (b, f // lane_tile)
    return pl.pallas_call(
        functools.partial(_bitonic_sort_kernel, n=n),
        grid=grid,
        in_specs=[pl.BlockSpec((1, n, lane_tile), lambda i, j: (i, 0, j))],
        out_specs=pl.BlockSpec((1, n, lane_tile), lambda i, j: (i, 0, j)),
        out_shape=jax.ShapeDtypeStruct(x.shape, x.dtype),
        compiler_params=pltpu.CompilerParams(
            dimension_semantics=("parallel", "parallel"),
        ),
    )(x)


# speedup vs baseline: 1.6249x; 1.0840x over previous
import functools

import jax
import jax.numpy as jnp
from jax.experimental import pallas as pl
from jax.experimental.pallas import tpu as pltpu

_C = 128
_LOGC = 7


def _ce_masked(v, k, j):
    C, L = v.shape
    d = 1 << j
    if d >= 8:
        G = C // (2 * d)
        w = v.reshape(G, 2, d, L)
        a = w[:, 0]
        b = w[:, 1]
        mn = jnp.minimum(a, b)
        mx = jnp.maximum(a, b)
        g = jax.lax.broadcasted_iota(jnp.int32, (G, 1, 1), 0)
        asc = ((g >> (k - j - 1)) & 1) == jnp.zeros((G, 1, 1), jnp.int32)
        na = jnp.where(asc, mn, mx)
        nb = jnp.where(asc, mx, mn)
        return jnp.concatenate([na[:, None], nb[:, None]], axis=1).reshape(
            C, L)
    il = jax.lax.broadcasted_iota(jnp.int32, (C, L), 0)
    zero = jnp.zeros((C, L), jnp.int32)
    upper = (il & d) != zero
    asc = (il & (1 << k)) == zero
    keep_min = upper != asc
    fwd = pltpu.roll(v, C - d, 0)
    bwd = pltpu.roll(v, d, 0)
    partner = jnp.where(upper, bwd, fwd)
    return jnp.where(keep_min, jnp.minimum(v, partner),
                     jnp.maximum(v, partner))


def _ce_static(v, j, ascending):
    C, L = v.shape
    d = 1 << j
    if d >= 8:
        G = C // (2 * d)
        w = v.reshape(G, 2, d, L)
        a = w[:, 0]
        b = w[:, 1]
        mn = jnp.minimum(a, b)
        mx = jnp.maximum(a, b)
        na, nb = (mn, mx) if ascending else (mx, mn)
        return jnp.concatenate([na[:, None], nb[:, None]], axis=1).reshape(
            C, L)
    il = jax.lax.broadcasted_iota(jnp.int32, (C, L), 0)
    upper = (il & d) != jnp.zeros((C, L), jnp.int32)
    fwd = pltpu.roll(v, C - d, 0)
    bwd = pltpu.roll(v, d, 0)
    if ascending:
        return jnp.where(upper, jnp.maximum(v, bwd), jnp.minimum(v, fwd))
    return jnp.where(upper, jnp.minimum(v, bwd), jnp.maximum(v, fwd))


def _merge_tail(v, ascending):
    for j in range(_LOGC - 1, -1, -1):
        v = _ce_static(v, j, ascending)
    return v


def _bitonic_sort_kernel(x_ref, o_ref, *, n: int):
    log_n = n.bit_length() - 1
    nc = n // _C

    def phase_a(c, carry):
        v = x_ref[0, pl.ds(c * _C, _C), :]
        for k in range(1, _LOGC):
            for j in range(k - 1, -1, -1):
                v = _ce_masked(v, k, j)
        asc7 = (c & 1) == 0

        @pl.when(asc7)
        def _():
            o_ref[0, pl.ds(c * _C, _C), :] = _merge_tail(v, True)

        @pl.when(jnp.logical_not(asc7))
        def _():
            o_ref[0, pl.ds(c * _C, _C), :] = _merge_tail(v, False)

        return carry

    jax.lax.fori_loop(0, nc, phase_a, 0)

    cq = 32

    for k in range(_LOGC + 1, log_n + 1):
        js = list(range(k - 1, _LOGC - 1, -1))
        fused = [(js[i], js[i + 1]) for i in range(0, len(js) - 1, 2)]
        single = js[-1] if len(js) % 2 else None

        for j, j2 in fused:
            d = 1 << j
            d2 = 1 << j2
            logm = j2 - 5

            def big2(t, carry, k=k, j=j, d=d, d2=d2, logm=logm):
                g = t >> logm
                p = t - (g << logm)
                base = (g << (j + 1)) + p * cq
                a0 = o_ref[0, pl.ds(base, cq), :]
                a1 = o_ref[0, pl.ds(base + d2, cq), :]
                a2 = o_ref[0, pl.ds(base + d, cq), :]
                a3 = o_ref[0, pl.ds(base + d + d2, cq), :]
                mn02 = jnp.minimum(a0, a2)
                mx02 = jnp.maximum(a0, a2)
                mn13 = jnp.minimum(a1, a3)
                mx13 = jnp.maximum(a1, a3)
                o0 = jnp.minimum(mn02, mn13)
                o1 = jnp.maximum(mn02, mn13)
                o2 = jnp.minimum(mx02, mx13)
                o3 = jnp.maximum(mx02, mx13)
                if k == log_n:
                    s0, s1, s2, s3 = 0, d2, d, d + d2
                else:
                    asc = ((g >> (k - j - 1)) & 1) == 0
                    s0 = jnp.where(asc, 0, d + d2)
                    s1 = jnp.where(asc, d2, d)
                    s2 = jnp.where(asc, d, d2)
                    s3 = jnp.where(asc, d + d2, 0)
                o_ref[0, pl.ds(base + s0, cq), :] = o0
                o_ref[0, pl.ds(base + s1, cq), :] = o1
                o_ref[0, pl.ds(base + s2, cq), :] = o2
                o_ref[0, pl.ds(base + s3, cq), :] = o3
                return carry

            jax.lax.fori_loop(0, n // (4 * cq), big2, 0)

        if single is not None:
            j = single
            d = 1 << j
            logm = j - _LOGC

            def big1(t, carry, k=k, j=j, d=d, logm=logm):
                g = t >> logm
                p = t - (g << logm)
                lo = (g << (j + 1)) + p * _C
                a = o_ref[0, pl.ds(lo, _C), :]
                b = o_ref[0, pl.ds(lo + d, _C), :]
                mn = jnp.minimum(a, b)
                mx = jnp.maximum(a, b)
                if k == log_n:
                    smn, smx = 0, d
                else:
                    asc = ((g >> (k - j - 1)) & 1) == 0
                    smn = jnp.where(asc, 0, d)
                    smx = jnp.where(asc, d, 0)
                o_ref[0, pl.ds(lo + smn, _C), :] = mn
                o_ref[0, pl.ds(lo + smx, _C), :] = mx
                return carry

            jax.lax.fori_loop(0, n // (2 * _C), big1, 0)

        def tail(c, carry, k=k):
            v = o_ref[0, pl.ds(c * _C, _C), :]
            if k == log_n:
                o_ref[0, pl.ds(c * _C, _C), :] = _merge_tail(v, True)
            else:
                asc = ((c >> (k - _LOGC)) & 1) == 0

                @pl.when(asc)
                def _():
                    o_ref[0, pl.ds(c * _C, _C), :] = _merge_tail(v, True)

                @pl.when(jnp.logical_not(asc))
                def _():
                    o_ref[0, pl.ds(c * _C, _C), :] = _merge_tail(v, False)

            return carry

        jax.lax.fori_loop(0, nc, tail, 0)


@jax.jit
def kernel(x):
    b, n, f = x.shape
    lane_tile = 128
    grid = (b, f // lane_tile)
    return pl.pallas_call(
        functools.partial(_bitonic_sort_kernel, n=n),
        grid=grid,
        in_specs=[pl.BlockSpec((1, n, lane_tile), lambda i, j: (i, 0, j))],
        out_specs=pl.BlockSpec((1, n, lane_tile), lambda i, j: (i, 0, j)),
        out_shape=jax.ShapeDtypeStruct(x.shape, x.dtype),
        compiler_params=pltpu.CompilerParams(
            dimension_semantics=("parallel", "parallel"),
        ),
    )(x)


# speedup vs baseline: 2.4952x; 1.5356x over previous
import functools

import jax
import jax.numpy as jnp
from jax.experimental import pallas as pl
from jax.experimental.pallas import tpu as pltpu

_C = 128
_T = _C // 8
_LOGC = 7


def _make_masks(L):
    io = jax.lax.broadcasted_iota(jnp.int32, (8, L), 0)
    zero = jnp.zeros((8, L), jnp.int32)
    upper = {d: (io & d) != zero for d in (1, 2, 4)}
    keep = {}
    for k, j in ((1, 0), (2, 1), (2, 0)):
        asc = (io & (1 << k)) == zero
        keep[(k, j)] = upper[1 << j] != asc
    return upper, keep


def _intra_masked(t, k, j, upper, keep):
    d = 1 << j
    fwd = pltpu.roll(t, 8 - d, 0)
    bwd = pltpu.roll(t, d, 0)
    partner = jnp.where(upper[d], bwd, fwd)
    return jnp.where(keep[(k, j)], jnp.minimum(t, partner),
                     jnp.maximum(t, partner))


def _intra_static(t, j, ascending, upper):
    d = 1 << j
    fwd = pltpu.roll(t, 8 - d, 0)
    bwd = pltpu.roll(t, d, 0)
    if ascending:
        return jnp.where(upper[d], jnp.maximum(t, bwd), jnp.minimum(t, fwd))
    return jnp.where(upper[d], jnp.minimum(t, bwd), jnp.maximum(t, fwd))


def _pair_stage(ts, j, asc_of_pair):
    dt = 1 << (j - 3)
    for s in range(_T):
        if s & dt:
            continue
        a, b = ts[s], ts[s + dt]
        mn = jnp.minimum(a, b)
        mx = jnp.maximum(a, b)
        if asc_of_pair(s):
            ts[s], ts[s + dt] = mn, mx
        else:
            ts[s], ts[s + dt] = mx, mn


def _merge_tiles(ts, ascending, upper):
    ts = list(ts)
    for j in range(_LOGC - 1, 2, -1):
        _pair_stage(ts, j, lambda s: ascending)
    for j in (2, 1, 0):
        for s in range(_T):
            ts[s] = _intra_static(ts[s], j, ascending, upper)
    return ts


def _load_tiles(ref, base):
    return [ref[0, pl.ds(base + 8 * s, 8), :] for s in range(_T)]


def _store_tiles(ref, base, ts):
    for s in range(_T):
        ref[0, pl.ds(base + 8 * s, 8), :] = ts[s]


def _bitonic_sort_kernel(x_ref, o_ref, *, n: int):
    log_n = n.bit_length() - 1
    nc = n // _C
    L = x_ref.shape[2]

    def phase_a(c, carry):
        upper, keep = _make_masks(L)
        base = c * _C
        ts = _load_tiles(x_ref, base)
        for k in range(1, _LOGC):
            for j in range(k - 1, -1, -1):
                if j >= 3:
                    _pair_stage(ts, j,
                                lambda s, k=k: not (s >> (k - 3)) & 1)
                elif k >= 3:
                    for s in range(_T):
                        ts[s] = _intra_static(
                            ts[s], j, not (s >> (k - 3)) & 1, upper)
                else:
                    for s in range(_T):
                        ts[s] = _intra_masked(ts[s], k, j, upper, keep)
        asc7 = (c & 1) == 0

        @pl.when(asc7)
        def _():
            _store_tiles(o_ref, base, _merge_tiles(ts, True, upper))

        @pl.when(jnp.logical_not(asc7))
        def _():
            _store_tiles(o_ref, base, _merge_tiles(ts, False, upper))

        return carry

    jax.lax.fori_loop(0, nc, phase_a, 0)

    cq = 32

    for k in range(_LOGC + 1, log_n + 1):
        js = list(range(k - 1, _LOGC - 1, -1))
        fused = [(js[i], js[i + 1]) for i in range(0, len(js) - 1, 2)]
        single = js[-1] if len(js) % 2 else None

        for j, j2 in fused:
            d = 1 << j
            d2 = 1 << j2
            logm = j2 - 5

            def big2(t, carry, k=k, j=j, d=d, d2=d2, logm=logm):
                g = t >> logm
                p = t - (g << logm)
                base = (g << (j + 1)) + p * cq
                a0 = o_ref[0, pl.ds(base, cq), :]
                a1 = o_ref[0, pl.ds(base + d2, cq), :]
                a2 = o_ref[0, pl.ds(base + d, cq), :]
                a3 = o_ref[0, pl.ds(base + d + d2, cq), :]
                mn02 = jnp.minimum(a0, a2)
                mx02 = jnp.maximum(a0, a2)
                mn13 = jnp.minimum(a1, a3)
                mx13 = jnp.maximum(a1, a3)
                o0 = jnp.minimum(mn02, mn13)
                o1 = jnp.maximum(mn02, mn13)
                o2 = jnp.minimum(mx02, mx13)
                o3 = jnp.maximum(mx02, mx13)
                if k == log_n:
                    s0, s1, s2, s3 = 0, d2, d, d + d2
                else:
                    asc = ((g >> (k - j - 1)) & 1) == 0
                    s0 = jnp.where(asc, 0, d + d2)
                    s1 = jnp.where(asc, d2, d)
                    s2 = jnp.where(asc, d, d2)
                    s3 = jnp.where(asc, d + d2, 0)
                o_ref[0, pl.ds(base + s0, cq), :] = o0
                o_ref[0, pl.ds(base + s1, cq), :] = o1
                o_ref[0, pl.ds(base + s2, cq), :] = o2
                o_ref[0, pl.ds(base + s3, cq), :] = o3
                return carry

            jax.lax.fori_loop(0, n // (4 * cq), big2, 0)

        if single is not None:
            j = single
            d = 1 << j
            logm = j - _LOGC

            def big1(t, carry, k=k, j=j, d=d, logm=logm):
                g = t >> logm
                p = t - (g << logm)
                lo = (g << (j + 1)) + p * _C
                a = o_ref[0, pl.ds(lo, _C), :]
                b = o_ref[0, pl.ds(lo + d, _C), :]
                mn = jnp.minimum(a, b)
                mx = jnp.maximum(a, b)
                if k == log_n:
                    smn, smx = 0, d
                else:
                    asc = ((g >> (k - j - 1)) & 1) == 0
                    smn = jnp.where(asc, 0, d)
                    smx = jnp.where(asc, d, 0)
                o_ref[0, pl.ds(lo + smn, _C), :] = mn
                o_ref[0, pl.ds(lo + smx, _C), :] = mx
                return carry

            jax.lax.fori_loop(0, n // (2 * _C), big1, 0)

        def tail(c, carry, k=k):
            upper, _ = _make_masks(L)
            base = c * _C
            ts = _load_tiles(o_ref, base)
            if k == log_n:
                _store_tiles(o_ref, base, _merge_tiles(ts, True, upper))
            else:
                asc = ((c >> (k - _LOGC)) & 1) == 0

                @pl.when(asc)
                def _():
                    _store_tiles(o_ref, base, _merge_tiles(ts, True, upper))

                @pl.when(jnp.logical_not(asc))
                def _():
                    _store_tiles(o_ref, base, _merge_tiles(ts, False, upper))

            return carry

        jax.lax.fori_loop(0, nc, tail, 0)


@jax.jit
def kernel(x):
    b, n, f = x.shape
    lane_tile = 128
    grid = (b, f // lane_tile)
    return pl.pallas_call(
        functools.partial(_bitonic_sort_kernel, n=n),
        grid=grid,
        in_specs=[pl.BlockSpec((1, n, lane_tile), lambda i, j: (i, 0, j))],
        out_specs=pl.BlockSpec((1, n, lane_tile), lambda i, j: (i, 0, j)),
        out_shape=jax.ShapeDtypeStruct(x.shape, x.dtype),
        compiler_params=pltpu.CompilerParams(
            dimension_semantics=("parallel", "parallel"),
        ),
    )(x)


# speedup vs baseline: 2.8507x; 1.1425x over previous
import functools

import jax
import jax.numpy as jnp
from jax.experimental import pallas as pl
from jax.experimental.pallas import tpu as pltpu

_C = 256
_T = _C // 8
_LOGC = 8


def _make_masks(L):
    io = jax.lax.broadcasted_iota(jnp.int32, (8, L), 0)
    zero = jnp.zeros((8, L), jnp.int32)
    upper = {d: (io & d) != zero for d in (1, 2, 4)}
    keep = {}
    for k, j in ((1, 0), (2, 1), (2, 0)):
        asc = (io & (1 << k)) == zero
        keep[(k, j)] = upper[1 << j] != asc
    return upper, keep


def _intra_masked(t, k, j, upper, keep):
    d = 1 << j
    fwd = pltpu.roll(t, 8 - d, 0)
    bwd = pltpu.roll(t, d, 0)
    partner = jnp.where(upper[d], bwd, fwd)
    return jnp.where(keep[(k, j)], jnp.minimum(t, partner),
                     jnp.maximum(t, partner))


def _intra_static(t, j, ascending, upper):
    d = 1 << j
    fwd = pltpu.roll(t, 8 - d, 0)
    bwd = pltpu.roll(t, d, 0)
    if ascending:
        return jnp.where(upper[d], jnp.maximum(t, bwd), jnp.minimum(t, fwd))
    return jnp.where(upper[d], jnp.minimum(t, bwd), jnp.maximum(t, fwd))


def _pair_stage(ts, j, asc_of_pair):
    dt = 1 << (j - 3)
    for s in range(_T):
        if s & dt:
            continue
        a, b = ts[s], ts[s + dt]
        mn = jnp.minimum(a, b)
        mx = jnp.maximum(a, b)
        if asc_of_pair(s):
            ts[s], ts[s + dt] = mn, mx
        else:
            ts[s], ts[s + dt] = mx, mn


def _merge_tiles(ts, ascending, upper):
    ts = list(ts)
    for j in range(_LOGC - 1, 2, -1):
        _pair_stage(ts, j, lambda s: ascending)
    for j in (2, 1, 0):
        for s in range(_T):
            ts[s] = _intra_static(ts[s], j, ascending, upper)
    return ts


def _load_tiles(ref, base):
    return [ref[0, pl.ds(base + 8 * s, 8), :] for s in range(_T)]


def _store_tiles(ref, base, ts):
    for s in range(_T):
        ref[0, pl.ds(base + 8 * s, 8), :] = ts[s]


def _bitonic_sort_kernel(x_ref, o_ref, *, n: int):
    log_n = n.bit_length() - 1
    nc = n // _C
    L = x_ref.shape[2]

    def phase_a(c, carry):
        upper, keep = _make_masks(L)
        base = c * _C
        ts = _load_tiles(x_ref, base)
        for k in range(1, _LOGC):
            for j in range(k - 1, -1, -1):
                if j >= 3:
                    _pair_stage(ts, j,
                                lambda s, k=k: not (s >> (k - 3)) & 1)
                elif k >= 3:
                    for s in range(_T):
                        ts[s] = _intra_static(
                            ts[s], j, not (s >> (k - 3)) & 1, upper)
                else:
                    for s in range(_T):
                        ts[s] = _intra_masked(ts[s], k, j, upper, keep)
        asc7 = (c & 1) == 0

        @pl.when(asc7)
        def _():
            _store_tiles(o_ref, base, _merge_tiles(ts, True, upper))

        @pl.when(jnp.logical_not(asc7))
        def _():
            _store_tiles(o_ref, base, _merge_tiles(ts, False, upper))

        return carry

    jax.lax.fori_loop(0, nc, phase_a, 0)

    cq = 32

    for k in range(_LOGC + 1, log_n + 1):
        js = list(range(k - 1, _LOGC - 1, -1))
        fused = [(js[i], js[i + 1]) for i in range(0, len(js) - 1, 2)]
        single = js[-1] if len(js) % 2 else None

        for j, j2 in fused:
            d = 1 << j
            d2 = 1 << j2
            logm = j2 - 5

            def big2(t, carry, k=k, j=j, d=d, d2=d2, logm=logm):
                g = t >> logm
                p = t - (g << logm)
                base = (g << (j + 1)) + p * cq
                a0 = o_ref[0, pl.ds(base, cq), :]
                a1 = o_ref[0, pl.ds(base + d2, cq), :]
                a2 = o_ref[0, pl.ds(base + d, cq), :]
                a3 = o_ref[0, pl.ds(base + d + d2, cq), :]
                mn02 = jnp.minimum(a0, a2)
                mx02 = jnp.maximum(a0, a2)
                mn13 = jnp.minimum(a1, a3)
                mx13 = jnp.maximum(a1, a3)
                o0 = jnp.minimum(mn02, mn13)
                o1 = jnp.maximum(mn02, mn13)
                o2 = jnp.minimum(mx02, mx13)
                o3 = jnp.maximum(mx02, mx13)
                if k == log_n:
                    s0, s1, s2, s3 = 0, d2, d, d + d2
                else:
                    asc = ((g >> (k - j - 1)) & 1) == 0
                    s0 = jnp.where(asc, 0, d + d2)
                    s1 = jnp.where(asc, d2, d)
                    s2 = jnp.where(asc, d, d2)
                    s3 = jnp.where(asc, d + d2, 0)
                o_ref[0, pl.ds(base + s0, cq), :] = o0
                o_ref[0, pl.ds(base + s1, cq), :] = o1
                o_ref[0, pl.ds(base + s2, cq), :] = o2
                o_ref[0, pl.ds(base + s3, cq), :] = o3
                return carry

            jax.lax.fori_loop(0, n // (4 * cq), big2, 0)

        if single is not None:
            j = single
            d = 1 << j
            logm = j - _LOGC

            def big1(t, carry, k=k, j=j, d=d, logm=logm):
                g = t >> logm
                p = t - (g << logm)
                lo = (g << (j + 1)) + p * _C
                a = o_ref[0, pl.ds(lo, _C), :]
                b = o_ref[0, pl.ds(lo + d, _C), :]
                mn = jnp.minimum(a, b)
                mx = jnp.maximum(a, b)
                if k == log_n:
                    smn, smx = 0, d
                else:
                    asc = ((g >> (k - j - 1)) & 1) == 0
                    smn = jnp.where(asc, 0, d)
                    smx = jnp.where(asc, d, 0)
                o_ref[0, pl.ds(lo + smn, _C), :] = mn
                o_ref[0, pl.ds(lo + smx, _C), :] = mx
                return carry

            jax.lax.fori_loop(0, n // (2 * _C), big1, 0)

        def tail(c, carry, k=k):
            upper, _ = _make_masks(L)
            base = c * _C
            ts = _load_tiles(o_ref, base)
            if k == log_n:
                _store_tiles(o_ref, base, _merge_tiles(ts, True, upper))
            else:
                asc = ((c >> (k - _LOGC)) & 1) == 0

                @pl.when(asc)
                def _():
                    _store_tiles(o_ref, base, _merge_tiles(ts, True, upper))

                @pl.when(jnp.logical_not(asc))
                def _():
                    _store_tiles(o_ref, base, _merge_tiles(ts, False, upper))

            return carry

        jax.lax.fori_loop(0, nc, tail, 0)


@jax.jit
def kernel(x):
    b, n, f = x.shape
    lane_tile = 128
    grid = (b, f // lane_tile)
    return pl.pallas_call(
        functools.partial(_bitonic_sort_kernel, n=n),
        grid=grid,
        in_specs=[pl.BlockSpec((1, n, lane_tile), lambda i, j: (i, 0, j))],
        out_specs=pl.BlockSpec((1, n, lane_tile), lambda i, j: (i, 0, j)),
        out_shape=jax.ShapeDtypeStruct(x.shape, x.dtype),
        compiler_params=pltpu.CompilerParams(
            dimension_semantics=("parallel", "parallel"),
        ),
    )(x)


# speedup vs baseline: 2.9958x; 1.0509x over previous
import functools

import jax
import jax.numpy as jnp
from jax.experimental import pallas as pl
from jax.experimental.pallas import tpu as pltpu

_C = 256
_T = _C // 8
_LOGC = 8


def _make_masks(L):
    io = jax.lax.broadcasted_iota(jnp.int32, (8, L), 0)
    zero = jnp.zeros((8, L), jnp.int32)
    upper = {d: (io & d) != zero for d in (1, 2, 4)}
    keep = {}
    for k, j in ((1, 0), (2, 1), (2, 0)):
        asc = (io & (1 << k)) == zero
        keep[(k, j)] = upper[1 << j] != asc
    return upper, keep


def _intra_masked(t, k, j, upper, keep):
    d = 1 << j
    fwd = pltpu.roll(t, 8 - d, 0)
    bwd = pltpu.roll(t, d, 0)
    partner = jnp.where(upper[d], bwd, fwd)
    return jnp.where(keep[(k, j)], jnp.minimum(t, partner),
                     jnp.maximum(t, partner))


def _intra_static(t, j, ascending, upper):
    d = 1 << j
    fwd = pltpu.roll(t, 8 - d, 0)
    bwd = pltpu.roll(t, d, 0)
    if ascending:
        return jnp.where(upper[d], jnp.maximum(t, bwd), jnp.minimum(t, fwd))
    return jnp.where(upper[d], jnp.minimum(t, bwd), jnp.maximum(t, fwd))


def _pair_stage(ts, j, asc_of_pair):
    dt = 1 << (j - 3)
    for s in range(_T):
        if s & dt:
            continue
        a, b = ts[s], ts[s + dt]
        mn = jnp.minimum(a, b)
        mx = jnp.maximum(a, b)
        if asc_of_pair(s):
            ts[s], ts[s + dt] = mn, mx
        else:
            ts[s], ts[s + dt] = mx, mn


def _merge_tiles(ts, ascending, upper):
    ts = list(ts)
    for j in range(_LOGC - 1, 2, -1):
        _pair_stage(ts, j, lambda s: ascending)
    for j in (2, 1, 0):
        for s in range(_T):
            ts[s] = _intra_static(ts[s], j, ascending, upper)
    return ts


def _load_tiles(ref, base):
    return [ref[0, pl.ds(base + 8 * s, 8), :] for s in range(_T)]


def _store_tiles(ref, base, ts):
    for s in range(_T):
        ref[0, pl.ds(base + 8 * s, 8), :] = ts[s]


def _bitonic_sort_kernel(x_ref, o_ref, *, n: int):
    log_n = n.bit_length() - 1
    nc = n // _C
    L = x_ref.shape[2]

    def phase_a(c, carry):
        upper, keep = _make_masks(L)
        base = c * _C
        ts = _load_tiles(x_ref, base)
        for k in range(1, _LOGC):
            for j in range(k - 1, -1, -1):
                if j >= 3:
                    _pair_stage(ts, j,
                                lambda s, k=k: not (s >> (k - 3)) & 1)
                elif k >= 3:
                    for s in range(_T):
                        ts[s] = _intra_static(
                            ts[s], j, not (s >> (k - 3)) & 1, upper)
                else:
                    for s in range(_T):
                        ts[s] = _intra_masked(ts[s], k, j, upper, keep)
        asc7 = (c & 1) == 0

        @pl.when(asc7)
        def _():
            _store_tiles(o_ref, base, _merge_tiles(ts, True, upper))

        @pl.when(jnp.logical_not(asc7))
        def _():
            _store_tiles(o_ref, base, _merge_tiles(ts, False, upper))

        return carry

    jax.lax.fori_loop(0, nc, phase_a, 0)

    sr = 32
    d_lo = _C

    for k in range(_LOGC + 1, log_n + 1):
        q = k - _LOGC
        nt = 1 << q
        logm = d_lo.bit_length() - 1 - (sr.bit_length() - 1)
        mm = d_lo // sr

        def big(tt, carry, k=k, q=q, nt=nt, logm=logm, mm=mm):
            g = tt >> logm
            p = tt - (g << logm)
            base = (g << k) + p * sr
            s = [o_ref[0, pl.ds(base + t * d_lo, sr), :] for t in range(nt)]
            for jj in range(k - 1, _LOGC - 1, -1):
                dt = 1 << (jj - _LOGC)
                for t in range(nt):
                    if t & dt:
                        continue
                    a, b = s[t], s[t + dt]
                    s[t] = jnp.minimum(a, b)
                    s[t + dt] = jnp.maximum(a, b)
            if k == log_n:
                for t in range(nt):
                    o_ref[0, pl.ds(base + t * d_lo, sr), :] = s[t]
            else:
                asc = (g & 1) == 0
                for t in range(nt):
                    dst = jnp.where(asc, t * d_lo, (nt - 1 - t) * d_lo)
                    o_ref[0, pl.ds(base + dst, sr), :] = s[t]
            return carry

        jax.lax.fori_loop(0, n // (nt * sr), big, 0)

        def tail(c, carry, k=k):
            upper, _ = _make_masks(L)
            base = c * _C
            ts = _load_tiles(o_ref, base)
            if k == log_n:
                _store_tiles(o_ref, base, _merge_tiles(ts, True, upper))
            else:
                asc = ((c >> (k - _LOGC)) & 1) == 0

                @pl.when(asc)
                def _():
                    _store_tiles(o_ref, base, _merge_tiles(ts, True, upper))

                @pl.when(jnp.logical_not(asc))
                def _():
                    _store_tiles(o_ref, base, _merge_tiles(ts, False, upper))

            return carry

        jax.lax.fori_loop(0, nc, tail, 0)


@jax.jit
def kernel(x):
    b, n, f = x.shape
    lane_tile = 128
    grid = (b, f // lane_tile)
    return pl.pallas_call(
        functools.partial(_bitonic_sort_kernel, n=n),
        grid=grid,
        in_specs=[pl.BlockSpec((1, n, lane_tile), lambda i, j: (i, 0, j))],
        out_specs=pl.BlockSpec((1, n, lane_tile), lambda i, j: (i, 0, j)),
        out_shape=jax.ShapeDtypeStruct(x.shape, x.dtype),
        compiler_params=pltpu.CompilerParams(
            dimension_semantics=("parallel", "parallel"),
        ),
    )(x)
